# Initial kernel scaffold; baseline (speedup 1.0000x reference)
import jax
import jax.numpy as jnp
from jax.experimental import pallas as pl


def kernel(decoded, class_id):
    raise NotImplementedError("write your pallas kernel here")


# speedup vs baseline: 27.4668x; 27.4668x over previous
import functools

import jax
import jax.numpy as jnp
from jax import lax
from jax.experimental import pallas as pl
from jax.experimental.pallas import tpu as pltpu
from jax.experimental.pallas import tpu_sc as plsc

OUT_DIM = 1000
BATCH = 16384
CHANNEL = 512
HALF = CHANNEL // 2
LANES = 16

NUM_WORKERS = 32
ROWS_PER_WORKER = BATCH // NUM_WORKERS
R = 8
NUM_CHUNKS = ROWS_PER_WORKER // R


def _build():
    mesh = plsc.VectorSubcoreMesh(core_axis_name="c", subcore_axis_name="s")

    @functools.partial(
        pl.kernel,
        mesh=mesh,
        out_type=jax.ShapeDtypeStruct((BATCH * OUT_DIM,), jnp.float32),
        scratch_types=[
            pltpu.VMEM((R * CHANNEL,), jnp.float32),
            pltpu.VMEM((R * HALF,), jnp.int32),
            pltpu.VMEM((R * OUT_DIM,), jnp.float32),
        ],
        compiler_params=pltpu.CompilerParams(needs_layout_passes=False),
    )
    def run(dec_hbm, cid_hbm, out_hbm, dec_v, cid_v, out_v):
        cid = lax.axis_index("c")
        sid = lax.axis_index("s")
        wid = sid * 2 + cid

        zeros = jnp.zeros((LANES,), jnp.float32)

        def chunk_body(ci, _):
            base_row = wid * ROWS_PER_WORKER + ci * R
            pltpu.sync_copy(dec_hbm.at[pl.ds(base_row * CHANNEL, R * CHANNEL)], dec_v)
            pltpu.sync_copy(cid_hbm.at[pl.ds(base_row * HALF, R * HALF)], cid_v)

            def zero_body(j, _):
                out_v[pl.ds(j * LANES, LANES)] = zeros
                return ()

            lax.fori_loop(0, (R * OUT_DIM) // LANES, zero_body, (), unroll=4)

            for r in range(R):
                row_off = r * OUT_DIM
                for k in range(HALF // LANES):
                    ids = cid_v[pl.ds(r * HALF + k * LANES, LANES)]
                    a = dec_v[pl.ds(r * CHANNEL + k * LANES, LANES)]
                    b = dec_v[pl.ds(r * CHANNEL + HALF + k * LANES, LANES)]
                    plsc.addupdate_scatter(out_v, [ids + row_off], a + b)

            pltpu.sync_copy(out_v, out_hbm.at[pl.ds(base_row * OUT_DIM, R * OUT_DIM)])
            return ()

        lax.fori_loop(0, NUM_CHUNKS, chunk_body, ())

    return run


_RUN = _build()


@jax.jit
def kernel(decoded, class_id):
    dec_flat = decoded.reshape(-1)
    cid_flat = class_id.astype(jnp.int32).reshape(-1)
    out = _RUN(dec_flat, cid_flat)
    return out.reshape(BATCH, OUT_DIM)


# speedup vs baseline: 36.9845x; 1.3465x over previous
import functools

import jax
import jax.numpy as jnp
from jax import lax
from jax.experimental import pallas as pl
from jax.experimental.pallas import tpu as pltpu
from jax.experimental.pallas import tpu_sc as plsc

OUT_DIM = 1000
BATCH = 16384
CHANNEL = 512
HALF = CHANNEL // 2
LANES = 16

NUM_WORKERS = 32
ROWS_PER_WORKER = BATCH // NUM_WORKERS
R = 16
NUM_CHUNKS = ROWS_PER_WORKER // R
NB = 2
NUM_GROUPS = NUM_CHUNKS // NB


def _build():
    mesh = plsc.VectorSubcoreMesh(core_axis_name="c", subcore_axis_name="s")

    @functools.partial(
        pl.kernel,
        mesh=mesh,
        out_type=jax.ShapeDtypeStruct((BATCH * OUT_DIM,), jnp.float32),
        scratch_types=[
            pltpu.VMEM((R * CHANNEL,), jnp.float32),
            pltpu.VMEM((R * CHANNEL,), jnp.float32),
            pltpu.VMEM((R * HALF,), jnp.int32),
            pltpu.VMEM((R * HALF,), jnp.int32),
            pltpu.VMEM((R * OUT_DIM,), jnp.float32),
            pltpu.VMEM((R * OUT_DIM,), jnp.float32),
            pltpu.SemaphoreType.DMA,
            pltpu.SemaphoreType.DMA,
            pltpu.SemaphoreType.DMA,
            pltpu.SemaphoreType.DMA,
        ],
        compiler_params=pltpu.CompilerParams(needs_layout_passes=False),
    )
    def run(
        dec_hbm, cid_hbm, out_hbm,
        dec_v0, dec_v1, cid_v0, cid_v1, out_v0, out_v1,
        si0, si1, so0, so1,
    ):
        cid = lax.axis_index("c")
        sid = lax.axis_index("s")
        wid = sid * 2 + cid
        row0 = wid * ROWS_PER_WORKER

        dec_v = (dec_v0, dec_v1)
        cid_v = (cid_v0, cid_v1)
        out_v = (out_v0, out_v1)
        sem_in = (si0, si1)
        sem_out = (so0, so1)
        zeros = jnp.zeros((LANES,), jnp.float32)

        def in_desc(ci, b):
            base = row0 + ci * R
            return (
                pltpu.make_async_copy(
                    dec_hbm.at[pl.ds(base * CHANNEL, R * CHANNEL)],
                    dec_v[b],
                    sem_in[b],
                ),
                pltpu.make_async_copy(
                    cid_hbm.at[pl.ds(base * HALF, R * HALF)],
                    cid_v[b],
                    sem_in[b],
                ),
            )

        def out_desc(ci, b):
            base = row0 + ci * R
            return pltpu.make_async_copy(
                out_v[b],
                out_hbm.at[pl.ds(base * OUT_DIM, R * OUT_DIM)],
                sem_out[b],
            )

        for b in range(NB):
            d0, d1 = in_desc(b, b)
            d0.start()
            d1.start()

        def group_body(g, _):
            for b in range(NB):
                ci = g * NB + b
                d0, d1 = in_desc(ci, b)
                d0.wait()
                d1.wait()

                @pl.when(g > 0)
                def _():
                    out_desc(ci - NB, b).wait()

                def zero_body(j, _):
                    out_v[b][pl.ds(j * LANES, LANES)] = zeros
                    return ()

                lax.fori_loop(
                    0, (R * OUT_DIM) // LANES, zero_body, (), unroll=8
                )

                for r in range(R):
                    row_off = r * OUT_DIM
                    for k in range(HALF // LANES):
                        ids = cid_v[b][pl.ds(r * HALF + k * LANES, LANES)]
                        a = dec_v[b][pl.ds(r * CHANNEL + k * LANES, LANES)]
                        c2 = dec_v[b][
                            pl.ds(r * CHANNEL + HALF + k * LANES, LANES)
                        ]
                        plsc.addupdate_scatter(
                            out_v[b], [ids + row_off], a + c2
                        )

                out_desc(ci, b).start()

                @pl.when(g < NUM_GROUPS - 1)
                def _():
                    n0, n1 = in_desc(ci + NB, b)
                    n0.start()
                    n1.start()

            return ()

        lax.fori_loop(0, NUM_GROUPS, group_body, ())

        for b in range(NB):
            out_desc((NUM_GROUPS - 1) * NB + b, b).wait()

    return run


_RUN = _build()


@jax.jit
def kernel(decoded, class_id):
    dec_flat = decoded.reshape(-1)
    cid_flat = class_id.astype(jnp.int32).reshape(-1)
    out = _RUN(dec_flat, cid_flat)
    return out.reshape(BATCH, OUT_DIM)


# speedup vs baseline: 51.5986x; 1.3951x over previous
import functools

import jax
import jax.numpy as jnp
from jax import lax
from jax.experimental import pallas as pl
from jax.experimental.pallas import tpu as pltpu
from jax.experimental.pallas import tpu_sc as plsc

OUT_DIM = 1000
BATCH = 16384
CHANNEL = 512
HALF = CHANNEL // 2
LANES = 16

NUM_WORKERS = 32
ROWS_PER_WORKER = BATCH // NUM_WORKERS
R = 16
NUM_CHUNKS = ROWS_PER_WORKER // R
NB = 2
NUM_GROUPS = NUM_CHUNKS // NB


def _build():
    mesh = plsc.VectorSubcoreMesh(core_axis_name="c", subcore_axis_name="s")

    @functools.partial(
        pl.kernel,
        mesh=mesh,
        out_type=jax.ShapeDtypeStruct((BATCH, OUT_DIM), jnp.float32),
        scratch_types=[
            pltpu.VMEM((R, CHANNEL), jnp.float32),
            pltpu.VMEM((R, CHANNEL), jnp.float32),
            pltpu.VMEM((R, HALF), jnp.int32),
            pltpu.VMEM((R, HALF), jnp.int32),
            pltpu.VMEM((R, OUT_DIM), jnp.float32),
            pltpu.VMEM((R, OUT_DIM), jnp.float32),
            pltpu.SemaphoreType.DMA,
            pltpu.SemaphoreType.DMA,
            pltpu.SemaphoreType.DMA,
            pltpu.SemaphoreType.DMA,
        ],
        compiler_params=pltpu.CompilerParams(needs_layout_passes=False),
    )
    def run(
        dec_hbm, cid_hbm, out_hbm,
        dec_v0, dec_v1, cid_v0, cid_v1, out_v0, out_v1,
        si0, si1, so0, so1,
    ):
        cid = lax.axis_index("c")
        sid = lax.axis_index("s")
        wid = sid * 2 + cid
        row0 = wid * ROWS_PER_WORKER

        dec_v = (dec_v0, dec_v1)
        cid_v = (cid_v0, cid_v1)
        out_v = (out_v0, out_v1)
        sem_in = (si0, si1)
        sem_out = (so0, so1)
        zeros = jnp.zeros((LANES,), jnp.float32)

        def in_desc(ci, b):
            base = row0 + ci * R
            return (
                pltpu.make_async_copy(
                    dec_hbm.at[pl.ds(base, R)], dec_v[b], sem_in[b]
                ),
                pltpu.make_async_copy(
                    cid_hbm.at[pl.ds(base, R)], cid_v[b], sem_in[b]
                ),
            )

        def out_desc(ci, b):
            base = row0 + ci * R
            return pltpu.make_async_copy(
                out_v[b], out_hbm.at[pl.ds(base, R)], sem_out[b]
            )

        for b in range(NB):
            d0, d1 = in_desc(b, b)
            d0.start()
            d1.start()

        def group_body(g, _):
            for b in range(NB):
                ci = g * NB + b
                d0, d1 = in_desc(ci, b)
                d0.wait()
                d1.wait()

                @pl.when(g > 0)
                def _():
                    out_desc(ci - NB, b).wait()

                def zero_body(r, _):
                    for j in range(OUT_DIM // LANES):
                        out_v[b][r, pl.ds(j * LANES, LANES)] = zeros
                    out_v[b][r, pl.ds(OUT_DIM - LANES, LANES)] = zeros
                    return ()

                lax.fori_loop(0, R, zero_body, ())

                for r in range(R):
                    rvec = jnp.full((LANES,), r, jnp.int32)
                    for k in range(HALF // LANES):
                        ids = cid_v[b][r, pl.ds(k * LANES, LANES)]
                        a = dec_v[b][r, pl.ds(k * LANES, LANES)]
                        c2 = dec_v[b][r, pl.ds(HALF + k * LANES, LANES)]
                        plsc.addupdate_scatter(
                            out_v[b], [rvec, ids], a + c2
                        )

                out_desc(ci, b).start()

                @pl.when(g < NUM_GROUPS - 1)
                def _():
                    n0, n1 = in_desc(ci + NB, b)
                    n0.start()
                    n1.start()

            return ()

        lax.fori_loop(0, NUM_GROUPS, group_body, ())

        for b in range(NB):
            out_desc((NUM_GROUPS - 1) * NB + b, b).wait()

    return run


_RUN = _build()


@jax.jit
def kernel(decoded, class_id):
    out = _RUN(decoded, class_id.astype(jnp.int32))
    return out


# speedup vs baseline: 51.6757x; 1.0015x over previous
import functools

import jax
import jax.numpy as jnp
from jax import lax
from jax.experimental import pallas as pl
from jax.experimental.pallas import tpu as pltpu
from jax.experimental.pallas import tpu_sc as plsc

OUT_DIM = 1000
BATCH = 16384
CHANNEL = 512
HALF = CHANNEL // 2
LANES = 16

NUM_WORKERS = 32
ROWS_PER_WORKER = BATCH // NUM_WORKERS
R = 16
NUM_CHUNKS = ROWS_PER_WORKER // R
NB = 2
NUM_GROUPS = NUM_CHUNKS // NB


def _build():
    mesh = plsc.VectorSubcoreMesh(core_axis_name="c", subcore_axis_name="s")

    @functools.partial(
        pl.kernel,
        mesh=mesh,
        out_type=jax.ShapeDtypeStruct((BATCH, OUT_DIM), jnp.float32),
        scratch_types=[
            pltpu.VMEM((R, CHANNEL), jnp.float32),
            pltpu.VMEM((R, CHANNEL), jnp.float32),
            pltpu.VMEM((R, HALF), jnp.int32),
            pltpu.VMEM((R, HALF), jnp.int32),
            pltpu.VMEM((R, OUT_DIM), jnp.float32),
            pltpu.VMEM((R, OUT_DIM), jnp.float32),
            pltpu.SemaphoreType.DMA,
            pltpu.SemaphoreType.DMA,
            pltpu.SemaphoreType.DMA,
            pltpu.SemaphoreType.DMA,
        ],
        compiler_params=pltpu.CompilerParams(
            needs_layout_passes=False,
            disable_bounds_checks=True,
            disable_semaphore_checks=True,
        ),
    )
    def run(
        dec_hbm, cid_hbm, out_hbm,
        dec_v0, dec_v1, cid_v0, cid_v1, out_v0, out_v1,
        si0, si1, so0, so1,
    ):
        cid = lax.axis_index("c")
        sid = lax.axis_index("s")
        wid = sid * 2 + cid
        row0 = wid * ROWS_PER_WORKER

        dec_v = (dec_v0, dec_v1)
        cid_v = (cid_v0, cid_v1)
        out_v = (out_v0, out_v1)
        sem_in = (si0, si1)
        sem_out = (so0, so1)
        zeros = jnp.zeros((LANES,), jnp.float32)

        def in_desc(ci, b):
            base = row0 + ci * R
            return (
                pltpu.make_async_copy(
                    dec_hbm.at[pl.ds(base, R)], dec_v[b], sem_in[b]
                ),
                pltpu.make_async_copy(
                    cid_hbm.at[pl.ds(base, R)], cid_v[b], sem_in[b]
                ),
            )

        def out_desc(ci, b):
            base = row0 + ci * R
            return pltpu.make_async_copy(
                out_v[b], out_hbm.at[pl.ds(base, R)], sem_out[b]
            )

        for b in range(NB):
            d0, d1 = in_desc(b, b)
            d0.start()
            d1.start()

        def group_body(g, _):
            for b in range(NB):
                ci = g * NB + b
                d0, d1 = in_desc(ci, b)
                d0.wait()
                d1.wait()

                @pl.when(g > 0)
                def _():
                    out_desc(ci - NB, b).wait()

                def zero_body(r, _):
                    for j in range(OUT_DIM // LANES):
                        out_v[b][r, pl.ds(j * LANES, LANES)] = zeros
                    out_v[b][r, pl.ds(OUT_DIM - LANES, LANES)] = zeros
                    return ()

                lax.fori_loop(0, R, zero_body, ())

                for r in range(R):
                    rvec = jnp.full((LANES,), r, jnp.int32)
                    for k in range(HALF // LANES):
                        ids = cid_v[b][r, pl.ds(k * LANES, LANES)]
                        a = dec_v[b][r, pl.ds(k * LANES, LANES)]
                        c2 = dec_v[b][r, pl.ds(HALF + k * LANES, LANES)]
                        plsc.addupdate_scatter(
                            out_v[b], [rvec, ids], a + c2
                        )

                out_desc(ci, b).start()

                @pl.when(g < NUM_GROUPS - 1)
                def _():
                    n0, n1 = in_desc(ci + NB, b)
                    n0.start()
                    n1.start()

            return ()

        lax.fori_loop(0, NUM_GROUPS, group_body, ())

        for b in range(NB):
            out_desc((NUM_GROUPS - 1) * NB + b, b).wait()

    return run


_RUN = _build()


@jax.jit
def kernel(decoded, class_id):
    out = _RUN(decoded, class_id.astype(jnp.int32))
    return out
